# baseline (speedup 1.0000x reference)
import functools

import jax
import jax.numpy as jnp
from jax import lax
from jax.experimental import pallas as pl
from jax.experimental.pallas import tpu as pltpu
from jax.experimental.pallas import tpu_sc as plsc

BT = 128
NC, NS = 2, 16
NW = NC * NS


def _sc_dispatch(hidden, slot_of_token, tkw, t_pad):
    T, D = hidden.shape
    chunk = t_pad // NW
    n_tok_vecs = T // 16
    mesh = plsc.VectorSubcoreMesh(core_axis_name="c", subcore_axis_name="s")

    @functools.partial(
        pl.kernel,
        mesh=mesh,
        out_type=(
            jax.ShapeDtypeStruct((t_pad, D), jnp.float32),
            jax.ShapeDtypeStruct((t_pad,), jnp.float32),
        ),
        scratch_types=[
            pltpu.VMEM((T,), jnp.int32),
            pltpu.VMEM((T,), jnp.float32),
            pltpu.VMEM((chunk,), jnp.int32),
            pltpu.VMEM((chunk,), jnp.float32),
            pltpu.VMEM((chunk, D), jnp.float32),
            pltpu.SemaphoreType.DMA,
        ],
        compiler_params=pltpu.CompilerParams(needs_layout_passes=False),
    )
    def dispatch_k(hid_hbm, slots_hbm, tkw_hbm, xout_hbm, wout_hbm,
                   slots_v, tkw_v, idx_v, w_v, rows_v, sem):
        wid = lax.axis_index("s") * NC + lax.axis_index("c")
        base = wid * chunk
        pltpu.sync_copy(slots_hbm, slots_v)
        pltpu.sync_copy(tkw_hbm, tkw_v)
        lane = lax.iota(jnp.int32, 16)
        zeros16 = jnp.zeros((16,), jnp.float32)
        for j in range(chunk // 16):
            idx_v[pl.ds(j * 16, 16)] = lax.rem(base + j * 16 + lane, T)
            w_v[pl.ds(j * 16, 16)] = zeros16
        for j in range(n_tok_vecs):
            s = slots_v[pl.ds(j * 16, 16)]
            m = (s >= base) & (s < base + chunk)
            loc = jnp.where(m, s - base, 0)
            plsc.store_scatter(idx_v, [loc], j * 16 + lane, mask=m)
            plsc.store_scatter(w_v, [loc], tkw_v[pl.ds(j * 16, 16)], mask=m)
        pltpu.async_copy(hid_hbm.at[idx_v], rows_v, sem).wait()
        pltpu.sync_copy(rows_v, xout_hbm.at[pl.ds(base, chunk)])
        pltpu.sync_copy(w_v, wout_hbm.at[pl.ds(base, chunk)])

    return dispatch_k(hidden, slot_of_token, tkw)


def _sc_combine_packed(y_pk, slot_of_token, out_rows, D):
    half = D // 2
    chunk = out_rows // NW
    mesh = plsc.VectorSubcoreMesh(core_axis_name="c", subcore_axis_name="s")

    @functools.partial(
        pl.kernel,
        mesh=mesh,
        out_type=jax.ShapeDtypeStruct((out_rows, D), jnp.float32),
        scratch_types=[
            pltpu.VMEM((chunk,), jnp.int32),
            pltpu.VMEM((chunk, half), jnp.int32),
            pltpu.VMEM((chunk, D), jnp.float32),
            pltpu.SemaphoreType.DMA,
        ],
        compiler_params=pltpu.CompilerParams(needs_layout_passes=False),
    )
    def combine_k(y_hbm, slots_hbm, out_hbm, idx_v, rows_pk, out_v, sem):
        wid = lax.axis_index("s") * NC + lax.axis_index("c")
        base = wid * chunk
        pltpu.sync_copy(slots_hbm.at[pl.ds(base, chunk)], idx_v)
        pltpu.async_copy(y_hbm.at[idx_v], rows_pk, sem).wait()
        hi_mask = jnp.full((16,), -65536, jnp.int32)

        def row_fn(r, carry):
            for c in range(half // 16):
                v = rows_pk[r, pl.ds(c * 16, 16)]
                lo = plsc.bitcast(lax.shift_left(v, 16), jnp.float32)
                hi = plsc.bitcast(v & hi_mask, jnp.float32)
                out_v[r, pl.ds(c * 16, 16)] = lo
                out_v[r, pl.ds(half + c * 16, 16)] = hi
            return carry

        lax.fori_loop(0, chunk, row_fn, 0)
        pltpu.sync_copy(out_v, out_hbm.at[pl.ds(base, chunk)])

    return combine_k(y_pk, slot_of_token)


def _sc_row_gather(table, idx, out_rows):
    D = table.shape[1]
    b_per_w = out_rows // NW
    chunk_rows = b_per_w
    n_chunks = 1
    mesh = plsc.VectorSubcoreMesh(core_axis_name="c", subcore_axis_name="s")

    @functools.partial(
        pl.kernel,
        mesh=mesh,
        out_type=jax.ShapeDtypeStruct((out_rows, D), jnp.float32),
        scratch_types=[
            pltpu.VMEM((chunk_rows,), jnp.int32),
            pltpu.VMEM((chunk_rows, D), jnp.float32),
            pltpu.SemaphoreType.DMA,
        ],
    )
    def gather_k(table_hbm, idx_hbm, out_hbm, idx_v, rows_v, sem):
        wid = lax.axis_index("s") * NC + lax.axis_index("c")
        base = wid * b_per_w
        for c in range(n_chunks):
            off = base + c * chunk_rows
            pltpu.sync_copy(idx_hbm.at[pl.ds(off, chunk_rows)], idx_v)
            pltpu.async_copy(table_hbm.at[idx_v], rows_v, sem).wait()
            pltpu.sync_copy(rows_v, out_hbm.at[pl.ds(off, chunk_rows)])

    return gather_k(table, idx)


def _moe_mlp_body(be_ref, bv_ref, bx_ref, x_ref, gu_ref, dn_ref, w_ref, o_ref):
    b = pl.program_id(0)
    inter = dn_ref.shape[2]

    @pl.when(bv_ref[b] == 1)
    def _():
        x = x_ref[...]
        gu = gu_ref[0]
        acc = lax.dot_general(x, gu, (((1,), (1,)), ((), ())),
                              preferred_element_type=jnp.float32)
        g = acc[:, :inter]
        u = acc[:, inter:]
        h = (g * jax.nn.sigmoid(g)) * u
        dn = dn_ref[0]
        y = lax.dot_general(h, dn, (((1,), (1,)), ((), ())),
                            preferred_element_type=jnp.float32)
        yw = y * w_ref[...]
        half = yw.shape[1] // 2
        lo_i = lax.bitcast_convert_type(yw[:, :half], jnp.int32)
        hi_i = lax.bitcast_convert_type(yw[:, half:], jnp.int32)
        rnd = jnp.int32(0x8000)
        lo_b = lax.shift_right_logical(lo_i + rnd, 16)
        hi_b = (hi_i + rnd) & jnp.int32(-65536)
        o_ref[...] = lo_b | hi_b


def _grouped_mlp(x_pad, gate_up_proj, down_proj, w_slot, block_expert,
                 block_valid, block_xidx):
    T_pad, H = x_pad.shape
    E, two_i, _ = gate_up_proj.shape
    inter = two_i // 2
    nb = T_pad // BT

    grid_spec = pltpu.PrefetchScalarGridSpec(
        num_scalar_prefetch=3,
        grid=(nb,),
        in_specs=[
            pl.BlockSpec((BT, H), lambda b, be, bv, bx: (bx[b], 0)),
            pl.BlockSpec((1, two_i, H), lambda b, be, bv, bx: (be[b], 0, 0)),
            pl.BlockSpec((1, H, inter), lambda b, be, bv, bx: (be[b], 0, 0)),
            pl.BlockSpec((BT, 1), lambda b, be, bv, bx: (bx[b], 0)),
        ],
        out_specs=pl.BlockSpec((BT, H // 2), lambda b, be, bv, bx: (bx[b], 0)),
    )
    return pl.pallas_call(
        _moe_mlp_body,
        grid_spec=grid_spec,
        out_shape=jax.ShapeDtypeStruct((T_pad, H // 2), jnp.int32),
    )(block_expert, block_valid, block_xidx, x_pad, gate_up_proj, down_proj,
      w_slot)


def _routing_metadata(top_k_index, top_k_weights, num_experts, t_pad):
    T = top_k_index.shape[0]
    e = top_k_index[:, 0].astype(jnp.int32)
    eids = jnp.arange(num_experts, dtype=jnp.int32)
    onehot = (e[:, None] == eids[None, :]).astype(jnp.int32)
    occ = jnp.cumsum(onehot, axis=0)
    rank = jnp.sum(onehot * occ, axis=1) - 1
    counts = occ[-1]
    aligned = ((counts + BT - 1) // BT) * BT
    ends = jnp.cumsum(aligned)
    starts = ends - aligned
    total_used = ends[-1]

    slot_of_token = jnp.sum(onehot * starts[None, :], axis=1) + rank

    nb = t_pad // BT
    bstarts = jnp.arange(nb, dtype=jnp.int32) * BT
    owner = jnp.minimum(
        jnp.sum((ends[None, :] <= bstarts[:, None]).astype(jnp.int32), axis=1),
        num_experts - 1)
    valid = (bstarts < total_used).astype(jnp.int32)
    last_owner = jnp.minimum(
        jnp.sum((ends <= total_used - 1).astype(jnp.int32)),
        num_experts - 1)
    block_expert = jnp.where(valid == 1, owner, last_owner).astype(jnp.int32)
    n_valid = total_used // BT
    block_xidx = jnp.where(valid == 1, jnp.arange(nb, dtype=jnp.int32),
                           n_valid - 1).astype(jnp.int32)
    return slot_of_token, block_expert, valid, block_xidx


def kernel(hidden_states, top_k_index, top_k_weights, gate_up_proj, down_proj):
    T, H = hidden_states.shape
    E = gate_up_proj.shape[0]
    t_pad = T + E * BT

    slot_of_token, block_expert, block_valid, block_xidx = (
        _routing_metadata(top_k_index, top_k_weights, E, t_pad))

    x_pad, w_flat = _sc_dispatch(hidden_states, slot_of_token,
                                 top_k_weights[:, 0], t_pad)
    y_pad = _grouped_mlp(x_pad, gate_up_proj, down_proj,
                         w_flat.reshape(t_pad, 1),
                         block_expert, block_valid, block_xidx)
    return _sc_combine_packed(y_pad, slot_of_token, T, H)


# speedup vs baseline: 1.0887x; 1.0887x over previous
import functools

import jax
import jax.numpy as jnp
from jax import lax
from jax.experimental import pallas as pl
from jax.experimental.pallas import tpu as pltpu
from jax.experimental.pallas import tpu_sc as plsc

BT = 128
NC, NS = 2, 16
NW = NC * NS


def _sc_dispatch(hidden, slot_of_token, tkw, t_pad):
    T, D = hidden.shape
    chunk = t_pad // NW
    n_tok_vecs = T // 16
    mesh = plsc.VectorSubcoreMesh(core_axis_name="c", subcore_axis_name="s")

    @functools.partial(
        pl.kernel,
        mesh=mesh,
        out_type=(
            jax.ShapeDtypeStruct((t_pad, D), jnp.float32),
            jax.ShapeDtypeStruct((t_pad, 1), jnp.float32),
        ),
        scratch_types=[
            pltpu.VMEM((T,), jnp.int32),
            pltpu.VMEM((T,), jnp.float32),
            pltpu.VMEM((chunk,), jnp.int32),
            pltpu.VMEM((chunk, 1), jnp.float32),
            pltpu.VMEM((chunk, D), jnp.float32),
            pltpu.SemaphoreType.DMA,
        ],
        compiler_params=pltpu.CompilerParams(needs_layout_passes=False),
    )
    def dispatch_k(hid_hbm, slots_hbm, tkw_hbm, xout_hbm, wout_hbm,
                   slots_v, tkw_v, idx_v, w_v, rows_v, sem):
        wid = lax.axis_index("s") * NC + lax.axis_index("c")
        base = wid * chunk
        pltpu.sync_copy(slots_hbm, slots_v)
        pltpu.sync_copy(tkw_hbm, tkw_v)
        lane = lax.iota(jnp.int32, 16)
        zeros16 = jnp.zeros((16,), jnp.float32)
        zcol = jnp.zeros((16,), jnp.int32)
        for j in range(chunk // 16):
            idx_v[pl.ds(j * 16, 16)] = lax.rem(base + j * 16 + lane, T)
            plsc.store_scatter(w_v, [j * 16 + lane, zcol], zeros16)
        for j in range(n_tok_vecs):
            s = slots_v[pl.ds(j * 16, 16)]
            m = (s >= base) & (s < base + chunk)
            loc = jnp.where(m, s - base, 0)
            plsc.store_scatter(idx_v, [loc], j * 16 + lane, mask=m)
            plsc.store_scatter(w_v, [loc, zcol], tkw_v[pl.ds(j * 16, 16)],
                               mask=m)
        pltpu.async_copy(hid_hbm.at[idx_v], rows_v, sem).wait()
        pltpu.sync_copy(rows_v, xout_hbm.at[pl.ds(base, chunk)])
        pltpu.sync_copy(w_v, wout_hbm.at[pl.ds(base, chunk)])

    return dispatch_k(hidden, slot_of_token, tkw)


def _sc_combine_packed(y_pk, slot_of_token, out_rows, D):
    half = D // 2
    chunk = out_rows // NW
    mesh = plsc.VectorSubcoreMesh(core_axis_name="c", subcore_axis_name="s")

    @functools.partial(
        pl.kernel,
        mesh=mesh,
        out_type=jax.ShapeDtypeStruct((out_rows, D), jnp.float32),
        scratch_types=[
            pltpu.VMEM((chunk,), jnp.int32),
            pltpu.VMEM((chunk, half), jnp.int32),
            pltpu.VMEM((chunk, D), jnp.float32),
            pltpu.SemaphoreType.DMA,
        ],
        compiler_params=pltpu.CompilerParams(needs_layout_passes=False),
    )
    def combine_k(y_hbm, slots_hbm, out_hbm, idx_v, rows_pk, out_v, sem):
        wid = lax.axis_index("s") * NC + lax.axis_index("c")
        base = wid * chunk
        pltpu.sync_copy(slots_hbm.at[pl.ds(base, chunk)], idx_v)
        pltpu.async_copy(y_hbm.at[idx_v], rows_pk, sem).wait()
        hi_mask = jnp.full((16,), -65536, jnp.int32)

        def row_fn(r, carry):
            for c in range(half // 16):
                v = rows_pk[r, pl.ds(c * 16, 16)]
                lo = plsc.bitcast(lax.shift_left(v, 16), jnp.float32)
                hi = plsc.bitcast(v & hi_mask, jnp.float32)
                out_v[r, pl.ds(c * 16, 16)] = lo
                out_v[r, pl.ds(half + c * 16, 16)] = hi
            return carry

        lax.fori_loop(0, chunk, row_fn, 0)
        pltpu.sync_copy(out_v, out_hbm.at[pl.ds(base, chunk)])

    return combine_k(y_pk, slot_of_token)


def _sc_row_gather(table, idx, out_rows):
    D = table.shape[1]
    b_per_w = out_rows // NW
    chunk_rows = b_per_w
    n_chunks = 1
    mesh = plsc.VectorSubcoreMesh(core_axis_name="c", subcore_axis_name="s")

    @functools.partial(
        pl.kernel,
        mesh=mesh,
        out_type=jax.ShapeDtypeStruct((out_rows, D), jnp.float32),
        scratch_types=[
            pltpu.VMEM((chunk_rows,), jnp.int32),
            pltpu.VMEM((chunk_rows, D), jnp.float32),
            pltpu.SemaphoreType.DMA,
        ],
    )
    def gather_k(table_hbm, idx_hbm, out_hbm, idx_v, rows_v, sem):
        wid = lax.axis_index("s") * NC + lax.axis_index("c")
        base = wid * b_per_w
        for c in range(n_chunks):
            off = base + c * chunk_rows
            pltpu.sync_copy(idx_hbm.at[pl.ds(off, chunk_rows)], idx_v)
            pltpu.async_copy(table_hbm.at[idx_v], rows_v, sem).wait()
            pltpu.sync_copy(rows_v, out_hbm.at[pl.ds(off, chunk_rows)])

    return gather_k(table, idx)


def _moe_mlp_body(be_ref, bv_ref, bx_ref, x_ref, gu_ref, dn_ref, w_ref, o_ref):
    b = pl.program_id(0)
    inter = dn_ref.shape[2]

    @pl.when(bv_ref[b] == 1)
    def _():
        x = x_ref[...]
        gu = gu_ref[0]
        acc = lax.dot_general(x, gu, (((1,), (1,)), ((), ())),
                              preferred_element_type=jnp.float32)
        g = acc[:, :inter]
        u = acc[:, inter:]
        h = (g * jax.nn.sigmoid(g)) * u
        dn = dn_ref[0]
        y = lax.dot_general(h, dn, (((1,), (1,)), ((), ())),
                            preferred_element_type=jnp.float32)
        o_ref[...] = y * w_ref[...]


def _grouped_mlp(x_pad, gate_up_proj, down_proj, w_slot, block_expert,
                 block_valid, block_xidx):
    T_pad, H = x_pad.shape
    E, two_i, _ = gate_up_proj.shape
    inter = two_i // 2
    nb = T_pad // BT

    grid_spec = pltpu.PrefetchScalarGridSpec(
        num_scalar_prefetch=3,
        grid=(nb,),
        in_specs=[
            pl.BlockSpec((BT, H), lambda b, be, bv, bx: (bx[b], 0)),
            pl.BlockSpec((1, two_i, H), lambda b, be, bv, bx: (be[b], 0, 0)),
            pl.BlockSpec((1, H, inter), lambda b, be, bv, bx: (be[b], 0, 0)),
            pl.BlockSpec((BT, 1), lambda b, be, bv, bx: (bx[b], 0)),
        ],
        out_specs=pl.BlockSpec((BT, H), lambda b, be, bv, bx: (bx[b], 0)),
    )
    return pl.pallas_call(
        _moe_mlp_body,
        grid_spec=grid_spec,
        out_shape=jax.ShapeDtypeStruct((T_pad, H), jnp.float32),
    )(block_expert, block_valid, block_xidx, x_pad, gate_up_proj, down_proj,
      w_slot)


def _routing_metadata(top_k_index, top_k_weights, num_experts, t_pad):
    T = top_k_index.shape[0]
    e = top_k_index[:, 0].astype(jnp.int32)
    eids = jnp.arange(num_experts, dtype=jnp.int32)
    onehot = (e[:, None] == eids[None, :]).astype(jnp.int32)
    occ = jnp.cumsum(onehot, axis=0)
    rank = jnp.sum(onehot * occ, axis=1) - 1
    counts = occ[-1]
    aligned = ((counts + BT - 1) // BT) * BT
    ends = jnp.cumsum(aligned)
    starts = ends - aligned
    total_used = ends[-1]

    slot_of_token = jnp.sum(onehot * starts[None, :], axis=1) + rank

    nb = t_pad // BT
    bstarts = jnp.arange(nb, dtype=jnp.int32) * BT
    owner = jnp.minimum(
        jnp.sum((ends[None, :] <= bstarts[:, None]).astype(jnp.int32), axis=1),
        num_experts - 1)
    valid = (bstarts < total_used).astype(jnp.int32)
    last_owner = jnp.minimum(
        jnp.sum((ends <= total_used - 1).astype(jnp.int32)),
        num_experts - 1)
    block_expert = jnp.where(valid == 1, owner, last_owner).astype(jnp.int32)
    n_valid = total_used // BT
    block_xidx = jnp.where(valid == 1, jnp.arange(nb, dtype=jnp.int32),
                           n_valid - 1).astype(jnp.int32)
    return slot_of_token, block_expert, valid, block_xidx


def kernel(hidden_states, top_k_index, top_k_weights, gate_up_proj, down_proj):
    T, H = hidden_states.shape
    E = gate_up_proj.shape[0]
    t_pad = T + E * BT

    slot_of_token, block_expert, block_valid, block_xidx = (
        _routing_metadata(top_k_index, top_k_weights, E, t_pad))

    x_pad, w_flat = _sc_dispatch(hidden_states, slot_of_token,
                                 top_k_weights[:, 0], t_pad)
    y_pad = _grouped_mlp(x_pad, gate_up_proj, down_proj, w_flat,
                         block_expert, block_valid, block_xidx)
    return _sc_row_gather(y_pad, slot_of_token, T)


# speedup vs baseline: 1.1093x; 1.0189x over previous
import functools

import jax
import jax.numpy as jnp
from jax import lax
from jax.experimental import pallas as pl
from jax.experimental.pallas import tpu as pltpu
from jax.experimental.pallas import tpu_sc as plsc

BT = 128
NC, NS = 2, 16
NW = NC * NS


def _sc_dispatch(hidden, slot_of_token, tkw, t_pad):
    T, D = hidden.shape
    chunk = t_pad // NW
    n_tok_vecs = T // 16
    mesh = plsc.VectorSubcoreMesh(core_axis_name="c", subcore_axis_name="s")

    @functools.partial(
        pl.kernel,
        mesh=mesh,
        out_type=(
            jax.ShapeDtypeStruct((t_pad, D), jnp.float32),
            jax.ShapeDtypeStruct((t_pad, 1), jnp.float32),
        ),
        scratch_types=[
            pltpu.VMEM((T,), jnp.int32),
            pltpu.VMEM((T,), jnp.float32),
            pltpu.VMEM((chunk,), jnp.int32),
            pltpu.VMEM((chunk, 1), jnp.float32),
            pltpu.VMEM((chunk, D), jnp.float32),
            pltpu.SemaphoreType.DMA,
        ],
        compiler_params=pltpu.CompilerParams(needs_layout_passes=False),
    )
    def dispatch_k(hid_hbm, slots_hbm, tkw_hbm, xout_hbm, wout_hbm,
                   slots_v, tkw_v, idx_v, w_v, rows_v, sem):
        wid = lax.axis_index("s") * NC + lax.axis_index("c")
        base = wid * chunk
        pltpu.sync_copy(slots_hbm, slots_v)
        pltpu.sync_copy(tkw_hbm, tkw_v)
        lane = lax.iota(jnp.int32, 16)
        zeros16 = jnp.zeros((16,), jnp.float32)
        zcol = jnp.zeros((16,), jnp.int32)
        for j in range(chunk // 16):
            idx_v[pl.ds(j * 16, 16)] = lax.rem(base + j * 16 + lane, T)
            plsc.store_scatter(w_v, [j * 16 + lane, zcol], zeros16)
        for j in range(n_tok_vecs):
            s = slots_v[pl.ds(j * 16, 16)]
            m = (s >= base) & (s < base + chunk)
            loc = jnp.where(m, s - base, 0)
            plsc.store_scatter(idx_v, [loc], j * 16 + lane, mask=m)
            plsc.store_scatter(w_v, [loc, zcol], tkw_v[pl.ds(j * 16, 16)],
                               mask=m)
        pltpu.async_copy(hid_hbm.at[idx_v], rows_v, sem).wait()
        pltpu.sync_copy(rows_v, xout_hbm.at[pl.ds(base, chunk)])
        pltpu.sync_copy(w_v, wout_hbm.at[pl.ds(base, chunk)])

    return dispatch_k(hidden, slot_of_token, tkw)


def _sc_combine_packed(y_pk, slot_of_token, out_rows, D):
    half = D // 2
    chunk = out_rows // NW
    mesh = plsc.VectorSubcoreMesh(core_axis_name="c", subcore_axis_name="s")

    @functools.partial(
        pl.kernel,
        mesh=mesh,
        out_type=jax.ShapeDtypeStruct((out_rows, D), jnp.float32),
        scratch_types=[
            pltpu.VMEM((chunk,), jnp.int32),
            pltpu.VMEM((chunk, half), jnp.int32),
            pltpu.VMEM((chunk, D), jnp.float32),
            pltpu.SemaphoreType.DMA,
        ],
        compiler_params=pltpu.CompilerParams(needs_layout_passes=False),
    )
    def combine_k(y_hbm, slots_hbm, out_hbm, idx_v, rows_pk, out_v, sem):
        wid = lax.axis_index("s") * NC + lax.axis_index("c")
        base = wid * chunk
        pltpu.sync_copy(slots_hbm.at[pl.ds(base, chunk)], idx_v)
        pltpu.async_copy(y_hbm.at[idx_v], rows_pk, sem).wait()
        hi_mask = jnp.full((16,), -65536, jnp.int32)

        def row_fn(r, carry):
            for c in range(half // 16):
                v = rows_pk[r, pl.ds(c * 16, 16)]
                lo = plsc.bitcast(lax.shift_left(v, 16), jnp.float32)
                hi = plsc.bitcast(v & hi_mask, jnp.float32)
                out_v[r, pl.ds(c * 16, 16)] = lo
                out_v[r, pl.ds(half + c * 16, 16)] = hi
            return carry

        lax.fori_loop(0, chunk, row_fn, 0)
        pltpu.sync_copy(out_v, out_hbm.at[pl.ds(base, chunk)])

    return combine_k(y_pk, slot_of_token)


def _sc_row_gather(table, idx, out_rows):
    D = table.shape[1]
    b_per_w = out_rows // NW
    chunk_rows = b_per_w
    n_chunks = 1
    mesh = plsc.VectorSubcoreMesh(core_axis_name="c", subcore_axis_name="s")

    @functools.partial(
        pl.kernel,
        mesh=mesh,
        out_type=jax.ShapeDtypeStruct((out_rows, D), jnp.float32),
        scratch_types=[
            pltpu.VMEM((chunk_rows,), jnp.int32),
            pltpu.VMEM((chunk_rows, D), jnp.float32),
            pltpu.SemaphoreType.DMA,
        ],
    )
    def gather_k(table_hbm, idx_hbm, out_hbm, idx_v, rows_v, sem):
        wid = lax.axis_index("s") * NC + lax.axis_index("c")
        base = wid * b_per_w
        for c in range(n_chunks):
            off = base + c * chunk_rows
            pltpu.sync_copy(idx_hbm.at[pl.ds(off, chunk_rows)], idx_v)
            pltpu.async_copy(table_hbm.at[idx_v], rows_v, sem).wait()
            pltpu.sync_copy(rows_v, out_hbm.at[pl.ds(off, chunk_rows)])

    return gather_k(table, idx)


def _moe_mlp_body(be_ref, bv_ref, bx_ref, x_ref, gu_ref, dn_ref, w_ref, o_ref):
    b = pl.program_id(0)
    inter = dn_ref.shape[2]

    @pl.when(bv_ref[b] == 1)
    def _():
        x = x_ref[...]
        gu = gu_ref[0]
        acc = lax.dot_general(x, gu, (((1,), (1,)), ((), ())),
                              preferred_element_type=jnp.float32)
        g = acc[:, :inter]
        u = acc[:, inter:]
        h = (g * jax.nn.sigmoid(g)) * u
        dn = dn_ref[0]
        y = lax.dot_general(h, dn, (((1,), (1,)), ((), ())),
                            preferred_element_type=jnp.float32)
        o_ref[...] = y * w_ref[...]


def _grouped_mlp(x_pad, gate_up_proj, down_proj, w_slot, block_expert,
                 block_valid, block_xidx):
    T_pad, H = x_pad.shape
    E, two_i, _ = gate_up_proj.shape
    inter = two_i // 2
    nb = T_pad // BT

    grid_spec = pltpu.PrefetchScalarGridSpec(
        num_scalar_prefetch=3,
        grid=(nb,),
        in_specs=[
            pl.BlockSpec((BT, H), lambda b, be, bv, bx: (bx[b], 0)),
            pl.BlockSpec((1, two_i, H), lambda b, be, bv, bx: (be[b], 0, 0)),
            pl.BlockSpec((1, H, inter), lambda b, be, bv, bx: (be[b], 0, 0)),
            pl.BlockSpec((BT, 1), lambda b, be, bv, bx: (bx[b], 0)),
        ],
        out_specs=pl.BlockSpec((BT, H), lambda b, be, bv, bx: (bx[b], 0)),
    )
    return pl.pallas_call(
        _moe_mlp_body,
        grid_spec=grid_spec,
        out_shape=jax.ShapeDtypeStruct((T_pad, H), jnp.float32),
    )(block_expert, block_valid, block_xidx, x_pad, gate_up_proj, down_proj,
      w_slot)


def _routing_body(tki_ref, slot_ref, be_ref, bv_ref, bx_ref):
    T = tki_ref.shape[0]
    E = 16
    nb = bx_ref.shape[0]
    e = tki_ref[...]
    eids = lax.broadcasted_iota(jnp.int32, (T, E), 1)
    onehot = (e == eids).astype(jnp.int32)
    row = lax.broadcasted_iota(jnp.int32, (T, E), 0)
    occ = onehot
    k = 1
    while k < T:
        sh = pltpu.roll(occ, k, 0)
        occ = occ + jnp.where(row >= k, sh, 0)
        k *= 2
    rank = jnp.sum(onehot * occ, axis=1) - 1
    counts = occ[T - 1:T, :]
    aligned = ((counts + BT - 1) // BT) * BT
    col = lax.broadcasted_iota(jnp.int32, (1, E), 1)
    ends = aligned
    k = 1
    while k < E:
        sh = pltpu.roll(ends, k, 1)
        ends = ends + jnp.where(col >= k, sh, 0)
        k *= 2
    starts = ends - aligned
    total_used = jnp.sum(jnp.where(col == E - 1, ends, 0))
    slot_ref[...] = jnp.sum(onehot * starts, axis=1) + rank

    bstarts = lax.broadcasted_iota(jnp.int32, (nb, E), 0) * BT
    ends_b = jnp.broadcast_to(ends, (nb, E))
    owner = jnp.minimum(jnp.sum((ends_b <= bstarts).astype(jnp.int32), axis=1),
                        E - 1)
    bids = lax.iota(jnp.int32, nb)
    valid = (bids * BT < total_used).astype(jnp.int32)
    last_owner = jnp.minimum(
        jnp.sum((ends <= total_used - 1).astype(jnp.int32)), E - 1)
    n_valid = total_used // BT
    be_ref[...] = jnp.where(valid == 1, owner, last_owner)
    bv_ref[...] = valid
    bx_ref[...] = jnp.where(valid == 1, bids, n_valid - 1)


def _routing_metadata_pallas(top_k_index, num_experts, t_pad):
    T = top_k_index.shape[0]
    nb = t_pad // BT
    return pl.pallas_call(
        _routing_body,
        out_shape=(
            jax.ShapeDtypeStruct((T,), jnp.int32),
            jax.ShapeDtypeStruct((nb,), jnp.int32),
            jax.ShapeDtypeStruct((nb,), jnp.int32),
            jax.ShapeDtypeStruct((nb,), jnp.int32),
        ),
    )(top_k_index)


def _routing_metadata(top_k_index, top_k_weights, num_experts, t_pad):
    T = top_k_index.shape[0]
    e = top_k_index[:, 0].astype(jnp.int32)
    eids = jnp.arange(num_experts, dtype=jnp.int32)
    onehot = (e[:, None] == eids[None, :]).astype(jnp.int32)
    occ = jnp.cumsum(onehot, axis=0)
    rank = jnp.sum(onehot * occ, axis=1) - 1
    counts = occ[-1]
    aligned = ((counts + BT - 1) // BT) * BT
    ends = jnp.cumsum(aligned)
    starts = ends - aligned
    total_used = ends[-1]

    slot_of_token = jnp.sum(onehot * starts[None, :], axis=1) + rank

    nb = t_pad // BT
    bstarts = jnp.arange(nb, dtype=jnp.int32) * BT
    owner = jnp.minimum(
        jnp.sum((ends[None, :] <= bstarts[:, None]).astype(jnp.int32), axis=1),
        num_experts - 1)
    valid = (bstarts < total_used).astype(jnp.int32)
    last_owner = jnp.minimum(
        jnp.sum((ends <= total_used - 1).astype(jnp.int32)),
        num_experts - 1)
    block_expert = jnp.where(valid == 1, owner, last_owner).astype(jnp.int32)
    n_valid = total_used // BT
    block_xidx = jnp.where(valid == 1, jnp.arange(nb, dtype=jnp.int32),
                           n_valid - 1).astype(jnp.int32)
    return slot_of_token, block_expert, valid, block_xidx


def kernel(hidden_states, top_k_index, top_k_weights, gate_up_proj, down_proj):
    T, H = hidden_states.shape
    E = gate_up_proj.shape[0]
    t_pad = T + E * BT

    slot_of_token, block_expert, block_valid, block_xidx = (
        _routing_metadata_pallas(top_k_index.astype(jnp.int32), E, t_pad))

    x_pad, w_flat = _sc_dispatch(hidden_states, slot_of_token,
                                 top_k_weights[:, 0], t_pad)
    y_pad = _grouped_mlp(x_pad, gate_up_proj, down_proj, w_flat,
                         block_expert, block_valid, block_xidx)
    return _sc_row_gather(y_pad, slot_of_token, T)
